# baseline (speedup 1.0000x reference)
import functools

import jax
import jax.numpy as jnp
from jax import lax
from jax.experimental import pallas as pl
from jax.experimental.pallas import tpu as pltpu
from jax.experimental.pallas import tpu_sc as plsc

N = 10000
E = 320000
D = 128
H = 64
L = 3
G = 128

NH = N // 2
NC = 2
NS = 16
NW = NC * NS
C = 128
CH = 80
KF = 5
NGRP = CH // KF
E_PAD = NW * CH * C
N_PAD = 10240
ROWS_PER_TILE = N_PAD // NS


def _make_sc_segsum():
    mesh = plsc.VectorSubcoreMesh(core_axis_name="c", subcore_axis_name="s")

    @functools.partial(
        pl.kernel,
        out_type=jax.ShapeDtypeStruct((NC, N_PAD, H), jnp.bfloat16),
        mesh=mesh,
        scratch_types=[
            pltpu.VMEM((CH, C), jnp.int32),
            pltpu.VMEM((CH, C), jnp.int32),
            pltpu.VMEM((KF, C, H), jnp.bfloat16),
            pltpu.VMEM_SHARED((N_PAD, H), jnp.bfloat16),
            pltpu.SemaphoreType.DMA((KF,)),
        ],
        compiler_params=pltpu.CompilerParams(use_tc_tiling_on_sc=False),
    )
    def sc_segsum(y_hbm, ei_hbm, zero_hbm, out_hbm,
                  src_v, dst_v, rows_v, agg_sh, sem):
        c = lax.axis_index("c")
        s = lax.axis_index("s")
        wid = s * NC + c
        row0 = pl.multiple_of(s * ROWS_PER_TILE, 8)

        pltpu.sync_copy(ei_hbm.at[0, pl.ds(wid * CH, CH)], src_v)
        pltpu.sync_copy(ei_hbm.at[1, pl.ds(wid * CH, CH)], dst_v)

        pltpu.sync_copy(zero_hbm.at[pl.ds(row0, ROWS_PER_TILE)],
                        agg_sh.at[pl.ds(row0, ROWS_PER_TILE)])
        plsc.subcore_barrier()

        def fire(g, b):
            pltpu.async_copy(y_hbm.at[src_v.at[g * KF + b]], rows_v.at[b],
                             sem.at[b])

        def drain(g, b):
            pltpu.make_async_copy(y_hbm.at[src_v.at[g * KF + b]],
                                  rows_v.at[b], sem.at[b]).wait()
            pltpu.sync_copy(rows_v.at[b], agg_sh.at[dst_v.at[g * KF + b]],
                            add=True)

        for b in range(KF):
            fire(0, b)

        def group(g, _):
            for b in range(KF):
                drain(g - 1, b)
                fire(g, b)
            return 0

        lax.fori_loop(1, NGRP, group, 0)
        for b in range(KF):
            drain(NGRP - 1, b)
        plsc.subcore_barrier()

        pltpu.sync_copy(agg_sh.at[pl.ds(row0, ROWS_PER_TILE)],
                        out_hbm.at[c, pl.ds(row0, ROWS_PER_TILE)])

    return sc_segsum


_sc_segsum = _make_sc_segsum()


def _mm0_body(x_ref, w_ref, o_ref, ybf_ref):
    a = jnp.dot(x_ref[:NH], w_ref[...], preferred_element_type=jnp.float32)
    b = jnp.dot(x_ref[NH:], w_ref[...], preferred_element_type=jnp.float32)
    o_ref[...] = jnp.concatenate([a, b], axis=1)
    ybf_ref[...] = jnp.concatenate(
        [a.astype(jnp.bfloat16), b.astype(jnp.bfloat16)], axis=0)


def _pool(h5, bf_ref, bs_ref):
    iota = lax.broadcasted_iota(jnp.int32, (G, NH), 0)
    ohf = (iota == bf_ref[...]).astype(jnp.float32)
    ohs = (iota == bs_ref[...]).astype(jnp.float32)
    m1 = jnp.dot(ohf, h5, preferred_element_type=jnp.float32)
    m2 = jnp.dot(ohs, h5, preferred_element_type=jnp.float32)
    return m1[:, :H] + m2[:, H:]


def _agg_pair(agg_ref):
    a0 = (agg_ref[0, :NH].astype(jnp.float32)
          + agg_ref[1, :NH].astype(jnp.float32))
    a1 = (agg_ref[0, NH:N].astype(jnp.float32)
          + agg_ref[1, NH:N].astype(jnp.float32))
    return jnp.concatenate([a0, a1], axis=1)


def _layer_body(y_ref, agg_ref, scale_ref, b1_ref, w2_ref, b2_ref,
                wn_ref, bf_ref, bs_ref, ynext_ref, ybf_ref, pooled_ref):
    t = scale_ref[...] * y_ref[...] + _agg_pair(agg_ref) + b1_ref[...]
    u = jnp.maximum(t, 0.0)
    h5 = jnp.maximum(
        jnp.dot(u, w2_ref[...], preferred_element_type=jnp.float32)
        + b2_ref[...], 0.0)
    yn = jnp.dot(h5, wn_ref[...], preferred_element_type=jnp.float32)
    ynext_ref[...] = yn
    ybf_ref[...] = jnp.concatenate(
        [yn[:, :H].astype(jnp.bfloat16), yn[:, H:].astype(jnp.bfloat16)],
        axis=0)
    pooled_ref[...] = _pool(h5, bf_ref, bs_ref)


def _layer_last_body(y_ref, agg_ref, scale_ref, b1_ref, w2_ref, b2_ref,
                     bf_ref, bs_ref, pooled_ref):
    t = scale_ref[...] * y_ref[...] + _agg_pair(agg_ref) + b1_ref[...]
    u = jnp.maximum(t, 0.0)
    h5 = jnp.maximum(
        jnp.dot(u, w2_ref[...], preferred_element_type=jnp.float32)
        + b2_ref[...], 0.0)
    pooled_ref[...] = _pool(h5, bf_ref, bs_ref)


def _blockdiag(w):
    z = jnp.zeros((H, H), jnp.float32)
    return jnp.concatenate(
        [jnp.concatenate([w, z], axis=1),
         jnp.concatenate([z, w], axis=1)], axis=0)


def _dup(b):
    return jnp.concatenate([b, b]).reshape(1, 2 * H)


def kernel(x, edge_index, batch, params, eps, center):
    npad = E_PAD - E
    k = jnp.arange(npad, dtype=jnp.int32)
    pads = jnp.stack([k % N, N + (k % (N_PAD - N))])
    ei = jnp.concatenate([edge_index, pads], axis=1).reshape(2, NW * CH, C)

    zeros = jnp.zeros((N_PAD, H), jnp.bfloat16)
    b2v = batch.reshape(2, NH)
    bf = b2v[0].reshape(1, NH)
    bs = b2v[1].reshape(1, NH)

    y5, ybf = pl.pallas_call(
        _mm0_body,
        out_shape=(jax.ShapeDtypeStruct((NH, 2 * H), jnp.float32),
                   jax.ShapeDtypeStruct((N, H), jnp.bfloat16)),
    )(x, params[0][0])

    pooled = []
    for l in range(L):
        W1, b1, W2, b2 = params[l]
        agg = _sc_segsum(ybf, ei, zeros)
        scale = (1.0 + eps[l]).reshape(1, 1)
        if l + 1 < L:
            y5, ybf, p = pl.pallas_call(
                _layer_body,
                out_shape=(jax.ShapeDtypeStruct((NH, 2 * H), jnp.float32),
                           jax.ShapeDtypeStruct((N, H), jnp.bfloat16),
                           jax.ShapeDtypeStruct((G, H), jnp.float32)),
            )(y5, agg, scale, _dup(b1), _blockdiag(W2), _dup(b2),
              _blockdiag(params[l + 1][0]), bf, bs)
        else:
            p = pl.pallas_call(
                _layer_last_body,
                out_shape=jax.ShapeDtypeStruct((G, H), jnp.float32),
            )(y5, agg, scale, _dup(b1), _blockdiag(W2), _dup(b2), bf, bs)
        pooled.append(p)

    z = jnp.concatenate(pooled, axis=-1)
    return (z, center)
